# Initial kernel scaffold; baseline (speedup 1.0000x reference)
import jax
import jax.numpy as jnp
from jax.experimental import pallas as pl


def kernel(h_v, edge_index, batch, W1, b1, W2, b2, W3, b3):
    raise NotImplementedError("write your pallas kernel here")


# speedup vs baseline: 13.6423x; 13.6423x over previous
import jax
import jax.numpy as jnp
from jax.experimental import pallas as pl

N_NODES = 10000
HIDDEN = 128
OUT_DIM = 1
NUM_GRAPHS = 256


def _readout_kernel(batch_ref, hv_ref, w1_ref, b1_ref, w2_ref, b2_ref,
                    w3_ref, b3_ref, out_ref):
    seg = jax.lax.broadcasted_iota(jnp.int32, (NUM_GRAPHS, 1), 0)
    onehot = (batch_ref[...] == seg).astype(jnp.float32)
    sums = jnp.dot(onehot, hv_ref[...], preferred_element_type=jnp.float32)
    counts = jnp.sum(onehot, axis=1, keepdims=True)
    pooled = sums / jnp.maximum(counts, 1.0)
    x = jnp.maximum(
        jnp.dot(pooled, w1_ref[...], preferred_element_type=jnp.float32)
        + b1_ref[...], 0.0)
    x = jnp.maximum(
        jnp.dot(x, w2_ref[...], preferred_element_type=jnp.float32)
        + b2_ref[...], 0.0)
    pred = jnp.dot(x, w3_ref[...], preferred_element_type=jnp.float32) + b3_ref[...]
    out_ref[...] = jnp.where(counts > 0.0, pred, 0.0)


def kernel(h_v, edge_index, batch, W1, b1, W2, b2, W3, b3):
    del edge_index
    batch2d = batch.astype(jnp.int32).reshape(1, N_NODES)
    return pl.pallas_call(
        _readout_kernel,
        out_shape=jax.ShapeDtypeStruct((NUM_GRAPHS, OUT_DIM), jnp.float32),
    )(batch2d, h_v, W1, b1.reshape(1, HIDDEN), W2, b2.reshape(1, HIDDEN),
      W3, b3.reshape(1, OUT_DIM))
